# baseline (speedup 1.0000x reference)
import functools

import jax
import jax.numpy as jnp
from jax import lax
from jax.experimental import pallas as pl
from jax.experimental.pallas import tpu as pltpu
from jax.experimental.pallas import tpu_sc as plsc

NC = 2
NS = 16
NW = NC * NS
L = 16


def _round_up(x, m):
    return (x + m - 1) // m * m


def _make_deg_kernel(epw, npad):
    mesh = plsc.VectorSubcoreMesh(core_axis_name="c", subcore_axis_name="s")

    @functools.partial(
        pl.kernel,
        mesh=mesh,
        out_type=jax.ShapeDtypeStruct((NW, npad), jnp.float32),
        compiler_params=pltpu.CompilerParams(needs_layout_passes=False),
        scratch_types=[
            pltpu.VMEM((epw,), jnp.int32),
            pltpu.VMEM((npad,), jnp.float32),
        ],
    )
    def deg_kernel(dst_hbm, out_hbm, idx_v, acc_v):
        wid = lax.axis_index("c") * NS + lax.axis_index("s")
        base = wid * epw
        pltpu.sync_copy(dst_hbm.at[pl.ds(base, epw)], idx_v)

        def zero_body(i, c):
            acc_v[pl.ds(pl.multiple_of(i * L, L), L)] = jnp.zeros((L,), jnp.float32)
            return c

        lax.fori_loop(0, npad // L, zero_body, 0)

        ones = jnp.ones((L,), jnp.float32)

        def body(i, c):
            idx = idx_v[pl.ds(pl.multiple_of(i * L, L), L)]
            plsc.addupdate_scatter(acc_v, [idx], ones)
            return c

        lax.fori_loop(0, epw // L, body, 0)
        pltpu.sync_copy(acc_v, out_hbm.at[wid])

    return deg_kernel


def _make_edge_kernel(epw, npad):
    mesh = plsc.VectorSubcoreMesh(core_axis_name="c", subcore_axis_name="s")

    @functools.partial(
        pl.kernel,
        mesh=mesh,
        out_type=(
            jax.ShapeDtypeStruct((NW, npad), jnp.float32),
            jax.ShapeDtypeStruct((NW, npad), jnp.float32),
        ),
        compiler_params=pltpu.CompilerParams(needs_layout_passes=False),
        scratch_types=[
            pltpu.VMEM((epw,), jnp.int32),
            pltpu.VMEM((epw,), jnp.int32),
            pltpu.VMEM((npad,), jnp.float32),
            pltpu.VMEM((npad,), jnp.float32),
            pltpu.VMEM((npad,), jnp.float32),
            pltpu.VMEM((npad,), jnp.float32),
        ],
    )
    def edge_kernel(src_hbm, dst_hbm, a1_hbm, a2_hbm, out1_hbm, out2_hbm,
                    src_v, dst_v, a1_v, a2_v, acc1_v, acc2_v):
        wid = lax.axis_index("c") * NS + lax.axis_index("s")
        base = wid * epw
        pltpu.sync_copy(src_hbm.at[pl.ds(base, epw)], src_v)
        pltpu.sync_copy(dst_hbm.at[pl.ds(base, epw)], dst_v)
        pltpu.sync_copy(a1_hbm, a1_v)
        pltpu.sync_copy(a2_hbm, a2_v)

        def zero_body(i, c):
            z = jnp.zeros((L,), jnp.float32)
            off = pl.ds(pl.multiple_of(i * L, L), L)
            acc1_v[off] = z
            acc2_v[off] = z
            return c

        lax.fori_loop(0, npad // L, zero_body, 0)

        def body(i, c):
            off = pl.ds(pl.multiple_of(i * L, L), L)
            s = src_v[off]
            d = dst_v[off]
            v1 = plsc.load_gather(a1_v, [s])
            v2 = plsc.load_gather(a2_v, [s])
            plsc.addupdate_scatter(acc1_v, [d], v1)
            plsc.addupdate_scatter(acc2_v, [d], v2)
            return c

        lax.fori_loop(0, epw // L, body, 0)
        pltpu.sync_copy(acc1_v, out1_hbm.at[wid])
        pltpu.sync_copy(acc2_v, out2_hbm.at[wid])

    return edge_kernel


def _bf(x):
    return x.astype(jnp.bfloat16).astype(jnp.float32)


def _h1_body(state_ref, wt_ref, out_ref):
    out_ref[...] = jnp.sum(_bf(state_ref[...]) * _bf(wt_ref[...]),
                           axis=1, keepdims=True)


def _run_h1(state, w_col):
    n, f = state.shape
    rb = 1000 if n % 1000 == 0 else n
    grid = n // rb
    return pl.pallas_call(
        _h1_body,
        grid=(grid,),
        in_specs=[
            pl.BlockSpec((rb, f), lambda i: (i, 0)),
            pl.BlockSpec((1, f), lambda i: (0, 0)),
        ],
        out_specs=pl.BlockSpec((rb, 1), lambda i: (i, 0)),
        out_shape=jax.ShapeDtypeStruct((n, 1), jnp.float32),
    )(state, w_col.reshape(1, f))


def _tables_body(wedge_ref, degp_ref, h1_ref, m_ref, s_ref, c_ref,
                 a1_ref, a2_ref, sl1_ref, sl2_ref, dinv_ref):
    deg = 1.0 + jnp.sum(degp_ref[...], axis=0)
    dinv = lax.rsqrt(deg)
    dinv2 = 1.0 / deg
    h1 = h1_ref[...]
    w0 = wedge_ref[0].astype(jnp.bfloat16).astype(jnp.float32)
    w1 = wedge_ref[1].astype(jnp.bfloat16).astype(jnp.float32)
    w2 = wedge_ref[2].astype(jnp.bfloat16).astype(jnp.float32)
    h2 = (_bf(m_ref[...]) * w0 + _bf(s_ref[...]) * w1 + _bf(c_ref[...]) * w2)
    a1_ref[...] = dinv * h1
    a2_ref[...] = dinv * h2
    sl1_ref[...] = dinv2 * h1
    sl2_ref[...] = dinv2 * h2
    dinv_ref[...] = dinv


def _run_tables(w_edge, deg_parts, h1p, massp, spinp, chargep, npad):
    rows = npad // 128
    rb = 8
    grid = rows // rb
    vec = jax.ShapeDtypeStruct((rows, 128), jnp.float32)
    blk = pl.BlockSpec((rb, 128), lambda i: (i, 0))
    return pl.pallas_call(
        _tables_body,
        grid=(grid,),
        in_specs=[
            pl.BlockSpec(memory_space=pltpu.SMEM),
            pl.BlockSpec((NW, rb, 128), lambda i: (0, i, 0)),
            blk, blk, blk, blk,
        ],
        out_specs=[blk, blk, blk, blk, blk],
        out_shape=[vec, vec, vec, vec, vec],
    )(w_edge, deg_parts, h1p, massp, spinp, chargep)


_NQ = 8


def _finish_body(bias_ref, qw_ref, out1_ref, out2_ref, dinv_ref,
                 sl1_ref, sl2_ref, w1_ref, w2_ref, y_ref, acc_ref):
    i = pl.program_id(0)
    nsteps = pl.num_programs(0)

    @pl.when(i == 0)
    def _():
        for j in range(8):
            acc_ref[j] = 0.0

    b_gcn_node = bias_ref[0]
    b_gcn_edge = bias_ref[1]
    g1 = _bf(dinv_ref[...] * jnp.sum(out1_ref[...], axis=0) + sl1_ref[...] + b_gcn_node)
    g2 = _bf(dinv_ref[...] * jnp.sum(out2_ref[...], axis=0) + sl2_ref[...] + b_gcn_edge)
    for j in range(4):
        acc_ref[j] = acc_ref[j] + jnp.sum(_bf(w1_ref[j]) * g1)
    for j in range(2):
        acc_ref[4 + j] = acc_ref[4 + j] + jnp.sum(_bf(w2_ref[j]) * g2)

    @pl.when(i == nsteps - 1)
    def _():
        for j in range(4):
            y_ref[j] = acc_ref[j] + bias_ref[2 + j]
        for j in range(2):
            y_ref[4 + j] = acc_ref[4 + j] + bias_ref[6 + j]
        y_ref[6] = bias_ref[8]
        y_ref[7] = bias_ref[9]


def _run_finish(bias, qw, out1, out2, dinv, sl1, sl2, w1p, w2p, npad):
    rows = npad // 128
    rb = 8
    grid = rows // rb
    blk = pl.BlockSpec((rb, 128), lambda i: (i, 0))
    return pl.pallas_call(
        _finish_body,
        grid=(grid,),
        in_specs=[
            pl.BlockSpec(memory_space=pltpu.SMEM),
            pl.BlockSpec(memory_space=pltpu.SMEM),
            pl.BlockSpec((NW, rb, 128), lambda i: (0, i, 0)),
            pl.BlockSpec((NW, rb, 128), lambda i: (0, i, 0)),
            blk, blk, blk,
            pl.BlockSpec((4, rb, 128), lambda i: (0, i, 0)),
            pl.BlockSpec((2, rb, 128), lambda i: (0, i, 0)),
        ],
        out_specs=pl.BlockSpec(memory_space=pltpu.SMEM),
        out_shape=jax.ShapeDtypeStruct((8,), jnp.float32),
        scratch_shapes=[pltpu.SMEM((8,), jnp.float32)],
    )(bias, qw, out1, out2, dinv, sl1, sl2, w1p, w2p)


def kernel(state, edge_index, mass, spin, charge, p_norm, theta,
           W_gcn_node, b_gcn_node, W_gcn_edge, b_gcn_edge,
           W_node_emb, b_node_emb, W_edge_emb, b_edge_emb, qnn_weights):
    n, f = state.shape
    e = edge_index.shape[1]
    npad = _round_up(n + 1, 1024)
    epad = _round_up(e, NW * L)
    epw = epad // NW

    src = edge_index[0].astype(jnp.int32)
    dst = edge_index[1].astype(jnp.int32)
    if epad != e:
        fill = jnp.full((epad - e,), n, jnp.int32)
        src = jnp.concatenate([src, fill])
        dst = jnp.concatenate([dst, fill])

    deg_parts = _make_deg_kernel(epw, npad)(dst)

    h1 = _run_h1(state, W_gcn_node.reshape(f)).reshape(n)

    def padv(v):
        return jnp.pad(v, (0, npad - n)).reshape(npad // 128, 128)

    a1, a2, sl1, sl2, dinv = _run_tables(
        W_gcn_edge.reshape(3), deg_parts.reshape(NW, npad // 128, 128),
        padv(h1), padv(mass), padv(spin), padv(charge), npad)

    out1, out2 = _make_edge_kernel(epw, npad)(
        src, dst, a1.reshape(npad), a2.reshape(npad))

    bias = jnp.concatenate([
        b_gcn_node.reshape(1), b_gcn_edge.reshape(1),
        b_node_emb.reshape(4), b_edge_emb.reshape(2),
        p_norm.reshape(1), theta.reshape(1)]).astype(jnp.float32)

    w1p = jnp.pad(W_node_emb, ((0, 0), (0, npad - n))).reshape(4, npad // 128, 128)
    w2p = jnp.pad(W_edge_emb, ((0, 0), (0, npad - n))).reshape(2, npad // 128, 128)

    feats = _run_finish(bias, qnn_weights.astype(jnp.float32),
                        out1.reshape(NW, npad // 128, 128),
                        out2.reshape(NW, npad // 128, 128),
                        dinv, sl1, sl2, w1p, w2p, npad)
    return _statevector_head(feats, qnn_weights)


def _sv_1q(state, gate, wire):
    state = jnp.tensordot(gate, state, axes=((1,), (wire,)))
    return jnp.moveaxis(state, 0, wire)


def _sv_cnot(state, ctrl, tgt):
    cnot = jnp.array([[1, 0, 0, 0], [0, 1, 0, 0], [0, 0, 0, 1], [0, 0, 1, 0]],
                     dtype=jnp.complex64).reshape(2, 2, 2, 2)
    state = jnp.tensordot(cnot, state, axes=((2, 3), (ctrl, tgt)))
    return jnp.moveaxis(state, (0, 1), (ctrl, tgt))


def _sv_rz(theta):
    e_m = jnp.exp(-0.5j * theta.astype(jnp.complex64))
    e_p = jnp.exp(0.5j * theta.astype(jnp.complex64))
    zero = jnp.zeros((), jnp.complex64)
    return jnp.stack([jnp.stack([e_m, zero]), jnp.stack([zero, e_p])])


def _sv_rx(theta):
    c = jnp.cos(theta / 2).astype(jnp.complex64)
    s = (-1j) * jnp.sin(theta / 2).astype(jnp.complex64)
    return jnp.stack([jnp.stack([c, s]), jnp.stack([s, c])])


def _statevector_head(features, weights):
    n = _NQ
    import numpy as _np
    state = jnp.zeros((2,) * n, jnp.complex64).at[(0,) * n].set(1.0)
    H = jnp.array([[1.0, 1.0], [1.0, -1.0]], jnp.complex64) / _np.sqrt(2.0)
    for i in range(n):
        state = _sv_1q(state, H, i)
    for i in range(n):
        state = _sv_1q(state, _sv_rz(features[i]), i)
    for i in range(n):
        state = _sv_cnot(state, i, (i + 1) % n)
    for b in range(weights.shape[0]):
        w = weights[b]
        for j in range(3):
            for i in range(n):
                state = _sv_1q(state, _sv_rx(w[i, j]), i)
                state = _sv_1q(state, _sv_rz(w[i, j]), i)
            for i in range(n):
                state = _sv_cnot(state, i, (i + 1) % n)
    probs = jnp.abs(state.reshape(2, -1)) ** 2
    return jnp.sum(probs[0]) - jnp.sum(probs[1])


# speedup vs baseline: 1.0008x; 1.0008x over previous
import functools

import numpy as np

import jax
import jax.numpy as jnp
from jax import lax
from jax.experimental import pallas as pl
from jax.experimental.pallas import tpu as pltpu
from jax.experimental.pallas import tpu_sc as plsc

NC = 2
NS = 16
NW = NC * NS
L = 16


def _round_up(x, m):
    return (x + m - 1) // m * m


def _make_deg_kernel(epw, npad):
    mesh = plsc.VectorSubcoreMesh(core_axis_name="c", subcore_axis_name="s")

    @functools.partial(
        pl.kernel,
        mesh=mesh,
        out_type=jax.ShapeDtypeStruct((NW, npad), jnp.float32),
        compiler_params=pltpu.CompilerParams(needs_layout_passes=False),
        scratch_types=[
            pltpu.VMEM((epw,), jnp.int32),
            pltpu.VMEM((npad,), jnp.float32),
        ],
    )
    def deg_kernel(dst_hbm, out_hbm, idx_v, acc_v):
        wid = lax.axis_index("c") * NS + lax.axis_index("s")
        base = wid * epw
        pltpu.sync_copy(dst_hbm.at[pl.ds(base, epw)], idx_v)

        def zero_body(i, c):
            acc_v[pl.ds(pl.multiple_of(i * L, L), L)] = jnp.zeros((L,), jnp.float32)
            return c

        lax.fori_loop(0, npad // L, zero_body, 0)

        ones = jnp.ones((L,), jnp.float32)

        def body(i, c):
            idx = idx_v[pl.ds(pl.multiple_of(i * L, L), L)]
            plsc.addupdate_scatter(acc_v, [idx], ones)
            return c

        lax.fori_loop(0, epw // L, body, 0)
        pltpu.sync_copy(acc_v, out_hbm.at[wid])

    return deg_kernel


def _make_edge_kernel(epw, npad):
    mesh = plsc.VectorSubcoreMesh(core_axis_name="c", subcore_axis_name="s")

    @functools.partial(
        pl.kernel,
        mesh=mesh,
        out_type=(
            jax.ShapeDtypeStruct((NW, npad), jnp.float32),
            jax.ShapeDtypeStruct((NW, npad), jnp.float32),
        ),
        compiler_params=pltpu.CompilerParams(needs_layout_passes=False),
        scratch_types=[
            pltpu.VMEM((epw,), jnp.int32),
            pltpu.VMEM((epw,), jnp.int32),
            pltpu.VMEM((npad,), jnp.float32),
            pltpu.VMEM((npad,), jnp.float32),
            pltpu.VMEM((npad,), jnp.float32),
            pltpu.VMEM((npad,), jnp.float32),
        ],
    )
    def edge_kernel(src_hbm, dst_hbm, a1_hbm, a2_hbm, out1_hbm, out2_hbm,
                    src_v, dst_v, a1_v, a2_v, acc1_v, acc2_v):
        wid = lax.axis_index("c") * NS + lax.axis_index("s")
        base = wid * epw
        pltpu.sync_copy(src_hbm.at[pl.ds(base, epw)], src_v)
        pltpu.sync_copy(dst_hbm.at[pl.ds(base, epw)], dst_v)
        pltpu.sync_copy(a1_hbm, a1_v)
        pltpu.sync_copy(a2_hbm, a2_v)

        def zero_body(i, c):
            z = jnp.zeros((L,), jnp.float32)
            off = pl.ds(pl.multiple_of(i * L, L), L)
            acc1_v[off] = z
            acc2_v[off] = z
            return c

        lax.fori_loop(0, npad // L, zero_body, 0)

        def body(i, c):
            off = pl.ds(pl.multiple_of(i * L, L), L)
            s = src_v[off]
            d = dst_v[off]
            v1 = plsc.load_gather(a1_v, [s])
            v2 = plsc.load_gather(a2_v, [s])
            plsc.addupdate_scatter(acc1_v, [d], v1)
            plsc.addupdate_scatter(acc2_v, [d], v2)
            return c

        lax.fori_loop(0, epw // L, body, 0)
        pltpu.sync_copy(acc1_v, out1_hbm.at[wid])
        pltpu.sync_copy(acc2_v, out2_hbm.at[wid])

    return edge_kernel


def _bf(x):
    return x.astype(jnp.bfloat16).astype(jnp.float32)


def _h1_body(state_ref, wt_ref, out_ref):
    out_ref[...] = jnp.sum(_bf(state_ref[...]) * _bf(wt_ref[...]),
                           axis=1, keepdims=True)


def _run_h1(state, w_col):
    n, f = state.shape
    rb = 1000 if n % 1000 == 0 else n
    grid = n // rb
    return pl.pallas_call(
        _h1_body,
        grid=(grid,),
        in_specs=[
            pl.BlockSpec((rb, f), lambda i: (i, 0)),
            pl.BlockSpec((1, f), lambda i: (0, 0)),
        ],
        out_specs=pl.BlockSpec((rb, 1), lambda i: (i, 0)),
        out_shape=jax.ShapeDtypeStruct((n, 1), jnp.float32),
    )(state, w_col.reshape(1, f))


def _tables_body(wedge_ref, degp_ref, h1_ref, m_ref, s_ref, c_ref,
                 a1_ref, a2_ref, sl1_ref, sl2_ref, dinv_ref):
    deg = 1.0 + jnp.sum(degp_ref[...], axis=0)
    dinv = lax.rsqrt(deg)
    dinv2 = 1.0 / deg
    h1 = h1_ref[...]
    w0 = wedge_ref[0].astype(jnp.bfloat16).astype(jnp.float32)
    w1 = wedge_ref[1].astype(jnp.bfloat16).astype(jnp.float32)
    w2 = wedge_ref[2].astype(jnp.bfloat16).astype(jnp.float32)
    h2 = (_bf(m_ref[...]) * w0 + _bf(s_ref[...]) * w1 + _bf(c_ref[...]) * w2)
    a1_ref[...] = dinv * h1
    a2_ref[...] = dinv * h2
    sl1_ref[...] = dinv2 * h1
    sl2_ref[...] = dinv2 * h2
    dinv_ref[...] = dinv


def _run_tables(w_edge, deg_parts, h1p, massp, spinp, chargep, npad):
    rows = npad // 128
    rb = 8
    grid = rows // rb
    vec = jax.ShapeDtypeStruct((rows, 128), jnp.float32)
    blk = pl.BlockSpec((rb, 128), lambda i: (i, 0))
    return pl.pallas_call(
        _tables_body,
        grid=(grid,),
        in_specs=[
            pl.BlockSpec(memory_space=pltpu.SMEM),
            pl.BlockSpec((NW, rb, 128), lambda i: (0, i, 0)),
            blk, blk, blk, blk,
        ],
        out_specs=[blk, blk, blk, blk, blk],
        out_shape=[vec, vec, vec, vec, vec],
    )(w_edge, deg_parts, h1p, massp, spinp, chargep)


_NQ = 8


def _finish_body(bias_ref, qw_ref, out1_ref, out2_ref, dinv_ref,
                 sl1_ref, sl2_ref, w1_ref, w2_ref, y_ref, acc_ref):
    i = pl.program_id(0)
    nsteps = pl.num_programs(0)

    @pl.when(i == 0)
    def _():
        for j in range(8):
            acc_ref[j] = 0.0

    b_gcn_node = bias_ref[0]
    b_gcn_edge = bias_ref[1]
    g1 = _bf(dinv_ref[...] * jnp.sum(out1_ref[...], axis=0) + sl1_ref[...] + b_gcn_node)
    g2 = _bf(dinv_ref[...] * jnp.sum(out2_ref[...], axis=0) + sl2_ref[...] + b_gcn_edge)
    for j in range(4):
        acc_ref[j] = acc_ref[j] + jnp.sum(_bf(w1_ref[j]) * g1)
    for j in range(2):
        acc_ref[4 + j] = acc_ref[4 + j] + jnp.sum(_bf(w2_ref[j]) * g2)

    @pl.when(i == nsteps - 1)
    def _():
        for j in range(4):
            y_ref[j] = acc_ref[j] + bias_ref[2 + j]
        for j in range(2):
            y_ref[4 + j] = acc_ref[4 + j] + bias_ref[6 + j]
        y_ref[6] = bias_ref[8]
        y_ref[7] = bias_ref[9]


def _run_finish(bias, qw, out1, out2, dinv, sl1, sl2, w1p, w2p, npad):
    rows = npad // 128
    rb = 8
    grid = rows // rb
    blk = pl.BlockSpec((rb, 128), lambda i: (i, 0))
    return pl.pallas_call(
        _finish_body,
        grid=(grid,),
        in_specs=[
            pl.BlockSpec(memory_space=pltpu.SMEM),
            pl.BlockSpec(memory_space=pltpu.SMEM),
            pl.BlockSpec((NW, rb, 128), lambda i: (0, i, 0)),
            pl.BlockSpec((NW, rb, 128), lambda i: (0, i, 0)),
            blk, blk, blk,
            pl.BlockSpec((4, rb, 128), lambda i: (0, i, 0)),
            pl.BlockSpec((2, rb, 128), lambda i: (0, i, 0)),
        ],
        out_specs=pl.BlockSpec(memory_space=pltpu.SMEM),
        out_shape=jax.ShapeDtypeStruct((8,), jnp.float32),
        scratch_shapes=[pltpu.SMEM((8,), jnp.float32)],
    )(bias, qw, out1, out2, dinv, sl1, sl2, w1p, w2p)


def kernel(state, edge_index, mass, spin, charge, p_norm, theta,
           W_gcn_node, b_gcn_node, W_gcn_edge, b_gcn_edge,
           W_node_emb, b_node_emb, W_edge_emb, b_edge_emb, qnn_weights):
    n, f = state.shape
    e = edge_index.shape[1]
    npad = _round_up(n + 1, 1024)
    epad = _round_up(e, NW * L)
    epw = epad // NW

    src = edge_index[0].astype(jnp.int32)
    dst = edge_index[1].astype(jnp.int32)
    if epad != e:
        fill = jnp.full((epad - e,), n, jnp.int32)
        src = jnp.concatenate([src, fill])
        dst = jnp.concatenate([dst, fill])

    deg_parts = _make_deg_kernel(epw, npad)(dst)

    h1 = _run_h1(state, W_gcn_node.reshape(f)).reshape(n)

    def padv(v):
        return jnp.pad(v, (0, npad - n)).reshape(npad // 128, 128)

    a1, a2, sl1, sl2, dinv = _run_tables(
        W_gcn_edge.reshape(3), deg_parts.reshape(NW, npad // 128, 128),
        padv(h1), padv(mass), padv(spin), padv(charge), npad)

    out1, out2 = _make_edge_kernel(epw, npad)(
        src, dst, a1.reshape(npad), a2.reshape(npad))

    bias = jnp.concatenate([
        b_gcn_node.reshape(1), b_gcn_edge.reshape(1),
        b_node_emb.reshape(4), b_edge_emb.reshape(2),
        p_norm.reshape(1), theta.reshape(1)]).astype(jnp.float32)

    w1p = jnp.pad(W_node_emb, ((0, 0), (0, npad - n))).reshape(4, npad // 128, 128)
    w2p = jnp.pad(W_edge_emb, ((0, 0), (0, npad - n))).reshape(2, npad // 128, 128)

    feats = _run_finish(bias, qnn_weights.astype(jnp.float32),
                        out1.reshape(NW, npad // 128, 128),
                        out2.reshape(NW, npad // 128, 128),
                        dinv, sl1, sl2, w1p, w2p, npad)
    return _statevector_head(feats, qnn_weights)


def _sv_1q(state, gate, wire):
    state = jnp.tensordot(gate, state, axes=((1,), (wire,)))
    return jnp.moveaxis(state, 0, wire)


def _sv_cnot(state, ctrl, tgt):
    cnot = jnp.array([[1, 0, 0, 0], [0, 1, 0, 0], [0, 0, 0, 1], [0, 0, 1, 0]],
                     dtype=jnp.complex64).reshape(2, 2, 2, 2)
    state = jnp.tensordot(cnot, state, axes=((2, 3), (ctrl, tgt)))
    return jnp.moveaxis(state, (0, 1), (ctrl, tgt))


def _sv_rz(theta):
    e_m = jnp.exp(-0.5j * theta.astype(jnp.complex64))
    e_p = jnp.exp(0.5j * theta.astype(jnp.complex64))
    zero = jnp.zeros((), jnp.complex64)
    return jnp.stack([jnp.stack([e_m, zero]), jnp.stack([zero, e_p])])


def _sv_rx(theta):
    c = jnp.cos(theta / 2).astype(jnp.complex64)
    s = (-1j) * jnp.sin(theta / 2).astype(jnp.complex64)
    return jnp.stack([jnp.stack([c, s]), jnp.stack([s, c])])


def _statevector_head(features, weights):
    n = _NQ
    state = jnp.zeros((2,) * n, jnp.complex64).at[(0,) * n].set(1.0)
    H = jnp.array([[1.0, 1.0], [1.0, -1.0]], jnp.complex64) / np.sqrt(2.0)
    for i in range(n):
        state = _sv_1q(state, H, i)
    for i in range(n):
        state = _sv_1q(state, _sv_rz(features[i]), i)
    for i in range(n):
        state = _sv_cnot(state, i, (i + 1) % n)
    for b in range(weights.shape[0]):
        w = weights[b]
        for j in range(3):
            for i in range(n):
                state = _sv_1q(state, _sv_rx(w[i, j]), i)
                state = _sv_1q(state, _sv_rz(w[i, j]), i)
            for i in range(n):
                state = _sv_cnot(state, i, (i + 1) % n)
    probs = jnp.abs(state.reshape(2, -1)) ** 2
    return jnp.sum(probs[0]) - jnp.sum(probs[1])
